# baseline (speedup 1.0000x reference)
import functools

import jax
import jax.numpy as jnp
from jax import lax
from jax.experimental import pallas as pl
from jax.experimental.pallas import tpu as pltpu
from jax.experimental.pallas import tpu_sc as plsc

N = 10000
F = 128
E = 320000

NC = 2
NS = 16
NW = NC * NS
EPW = E // NW
K = 32
NCHUNK = 320
EPT = K * NCHUNK
ROW_STRIDE = 624
ROW_COPY = 640


def _spmm_partials(h, packed, vals3, zeros):
    mesh = plsc.VectorSubcoreMesh(core_axis_name="c", subcore_axis_name="s")

    @functools.partial(
        pl.kernel,
        out_type=jax.ShapeDtypeStruct((NC, N, F), jnp.float32),
        mesh=mesh,
        scratch_types=[
            pltpu.VMEM((8, 2, K), jnp.int32),
            pltpu.VMEM((8, K), jnp.float32),
            pltpu.VMEM((4, K, F), jnp.float32),
            pltpu.VMEM_SHARED((N, F), jnp.float32),
            [pltpu.SemaphoreType.DMA] * 8,
            [pltpu.SemaphoreType.DMA] * 4,
            [pltpu.SemaphoreType.DMA] * 4,
        ],
    )
    def k(h_hbm, e_hbm, v_hbm, z_hbm, out_hbm,
          pbuf, vbuf, rows, acc_sh, isems, gsems, ssems):
        cid = lax.axis_index("c")
        sid = lax.axis_index("s")
        wid = cid * NS + sid

        rstart = pl.multiple_of(sid * ROW_STRIDE, 8)

        def start_idx(ci, q):
            pltpu.async_copy(e_hbm.at[wid, ci], pbuf.at[q], isems[q])
            pltpu.async_copy(v_hbm.at[wid, ci], vbuf.at[q], isems[q])

        def wait_idx(q):
            pltpu.make_async_copy(e_hbm.at[0, 0], pbuf.at[q],
                                  isems[q]).wait()
            pltpu.make_async_copy(v_hbm.at[0, 0], vbuf.at[q],
                                  isems[q]).wait()

        def start_gather(q, b):
            pltpu.async_copy(h_hbm.at[pbuf.at[q, 0]], rows.at[b], gsems[b])

        def wait_rows_bytes(sem, b):
            pltpu.make_async_copy(h_hbm.at[pl.ds(0, K)], rows.at[b],
                                  sem).wait()

        def chunk(ci, q, *, a=True, bf=True, c=True, g=True):
            b = q % 4
            q2, b2, q6 = (q + 2) % 8, (q + 2) % 4, (q + 6) % 8
            if a:
                wait_idx(q2)
            if bf:
                wait_rows_bytes(ssems[b2], b2)
            if c:
                start_gather(q2, b2)
            wait_rows_bytes(gsems[b], b)

            def scale(gi, c2):
                vvec = vbuf[q, pl.ds(16 * gi, 16)]
                for i in range(16):
                    v = vvec[i]
                    e = 16 * gi + i
                    for j in range(F // 16):
                        sl = pl.ds(16 * j, 16)
                        rows[b, e, sl] = rows[b, e, sl] * v
                return c2
            lax.fori_loop(0, K // 16, scale, 0)
            pltpu.async_copy(rows.at[b], acc_sh.at[pbuf.at[q, 1]], ssems[b],
                             add=True)
            if g:
                start_idx(ci + 6, q6)

        for q in range(6):
            start_idx(q, q)
        pltpu.sync_copy(z_hbm.at[pl.ds(rstart, ROW_COPY)],
                        acc_sh.at[pl.ds(rstart, ROW_COPY)])
        plsc.subcore_barrier()
        wait_idx(0)
        wait_idx(1)
        start_gather(0, 0)
        start_gather(1, 1)
        chunk(0, 0, bf=False)
        chunk(1, 1, bf=False)
        for ci in range(2, 8):
            chunk(ci, ci % 8)

        def octet(t, carry):
            base = 8 * t + 8
            for off in range(8):
                chunk(base + off, off)
            return carry
        lax.fori_loop(0, (NCHUNK - 16) // 8, octet, 0)

        for ci in range(NCHUNK - 8, NCHUNK):
            flags = dict(a=ci + 2 < NCHUNK, c=ci + 2 < NCHUNK,
                         g=ci + 6 < NCHUNK)
            chunk(ci, ci % 8, **flags)
        wait_rows_bytes(ssems[(NCHUNK - 2) % 4], (NCHUNK - 2) % 4)
        wait_rows_bytes(ssems[(NCHUNK - 1) % 4], (NCHUNK - 1) % 4)

        plsc.subcore_barrier()
        pltpu.sync_copy(acc_sh.at[pl.ds(rstart, ROW_COPY)],
                        out_hbm.at[cid, pl.ds(rstart, ROW_COPY)])

    return k(h, packed, vals3, zeros)


_BM = 1000


def _mm_in(x, w_t, b):
    def body(x_ref, w_ref, b_ref, o_ref):
        o_ref[...] = jnp.dot(x_ref[...], w_ref[...],
                             preferred_element_type=jnp.float32) + b_ref[...]
    return pl.pallas_call(
        body,
        grid=(N // _BM,),
        in_specs=[pl.BlockSpec((_BM, F), lambda i: (i, 0)),
                  pl.BlockSpec((F, F), lambda i: (0, 0)),
                  pl.BlockSpec((1, F), lambda i: (0, 0))],
        out_specs=pl.BlockSpec((_BM, F), lambda i: (i, 0)),
        out_shape=jax.ShapeDtypeStruct((N, F), jnp.float32),
    )(x, w_t, b.reshape(1, F))


def _combine_scale(parts, scal):
    def body(s_ref, p_ref, o_ref):
        s = p_ref[0] + p_ref[1]
        o_ref[...] = jnp.where(s > 0, s, (jnp.exp(s) - 1.0)) * s_ref[0]
    return pl.pallas_call(
        body,
        grid=(N // _BM,),
        in_specs=[pl.BlockSpec(memory_space=pltpu.SMEM),
                  pl.BlockSpec((NC, _BM, F), lambda i: (0, i, 0))],
        out_specs=pl.BlockSpec((_BM, F), lambda i: (i, 0)),
        out_shape=jax.ShapeDtypeStruct((N, F), jnp.float32),
    )(scal, parts)


def _combine_mm_out(parts, w_t, b):
    def body(p_ref, w_ref, b_ref, o_ref):
        s = p_ref[0] + p_ref[1]
        h = jnp.where(s > 0, s, (jnp.exp(s) - 1.0))
        o_ref[...] = jnp.dot(h, w_ref[...],
                             preferred_element_type=jnp.float32) + b_ref[...]
    return pl.pallas_call(
        body,
        grid=(N // _BM,),
        in_specs=[pl.BlockSpec((NC, _BM, F), lambda i: (0, i, 0)),
                  pl.BlockSpec((F, F), lambda i: (0, 0)),
                  pl.BlockSpec((1, F), lambda i: (0, 0))],
        out_specs=pl.BlockSpec((_BM, F), lambda i: (i, 0)),
        out_shape=jax.ShapeDtypeStruct((N, F), jnp.float32),
    )(parts, w_t, b.reshape(1, F))


def kernel(x, edge_index, edge_vals, W1, b1, scalar, Wout, bout):
    pad = ((0, 0), (0, EPT - EPW))
    src3 = jnp.pad(edge_index[1].reshape(NW, EPW), pad)
    dst3 = jnp.pad(edge_index[0].reshape(NW, EPW), pad)
    packed = jnp.concatenate([src3.reshape(NW, NCHUNK, 1, K),
                              dst3.reshape(NW, NCHUNK, 1, K)], axis=2)
    vals3 = jnp.pad(edge_vals.reshape(NW, EPW), pad).reshape(NW, NCHUNK, K)
    zeros = jnp.zeros((N, F), jnp.float32)

    h = _mm_in(x, W1.T, b1)
    parts = _spmm_partials(h, packed, vals3, zeros)
    for _ in range(2):
        g = _combine_scale(parts, scalar)
        parts = _spmm_partials(g, packed, vals3, zeros)
    return _combine_mm_out(parts, Wout.T, bout)
